# Initial kernel scaffold; baseline (speedup 1.0000x reference)
import jax
import jax.numpy as jnp
from jax.experimental import pallas as pl


def kernel(x, edge_index, edge_weight, W1, b1, W2, b2, bn1_g, bn1_b, bn2_g, bn2_b, Wih0, Whh0, bih0, bhh0, Wih1, Whh1, bih1, bhh1, attn_w_W, attn_w_b, attn_v_W, attn_v_b, fc1_W, fc1_b, fc2_W, fc2_b):
    raise NotImplementedError("write your pallas kernel here")


# speedup vs baseline: 2.7695x; 2.7695x over previous
import functools

import jax
import jax.numpy as jnp
from jax import lax
from jax.experimental import pallas as pl

N_NODES = 10000
WINDOW = 3
NFEAT = 128
NHID = 128
N_TOTAL = N_NODES * WINDOW
N_EDGES = 480000

_BLK = 400
_NBLK = N_NODES // _BLK


def _fused_dense_body(h1t0, h1t1, h1t2, h2t0, h2t1, h2t2, xt0, xt1, xt2,
                      Wih0, Whh0, b0, Wih1, Whh1, b1v,
                      awW, awb, avW, avb, f1W, f1b, f2W, f2b, out):
    def mm(a, w):
        return lax.dot_general(a, w[...], (((1,), (1,)), ((), ())),
                               preferred_element_type=jnp.float32)

    xs = [jnp.concatenate([h1t0[...], h2t0[...]], axis=1),
          jnp.concatenate([h1t1[...], h2t1[...]], axis=1),
          jnp.concatenate([h1t2[...], h2t2[...]], axis=1)]

    def lstm(inputs, Wih, Whh, b):
        h = jnp.zeros((inputs[0].shape[0], NHID), jnp.float32)
        c = h
        ys = []
        for t in range(WINDOW):
            g = mm(inputs[t], Wih) + mm(h, Whh) + b[...]
            i, f, gg, o = jnp.split(g, 4, axis=-1)
            c = jax.nn.sigmoid(f) * c + jax.nn.sigmoid(i) * jnp.tanh(gg)
            h = jax.nn.sigmoid(o) * jnp.tanh(c)
            ys.append(h)
        return ys

    ys0 = lstm(xs, Wih0, Whh0, b0)
    ys1 = lstm(ys0, Wih1, Whh1, b1v)

    scores = []
    for t in range(WINDOW):
        a = jnp.tanh(mm(ys1[t], awW) + awb[...])
        scores.append(jnp.sum(a * avW[...], axis=1, keepdims=True)
                      + avb[0, 0])
    m = jnp.maximum(jnp.maximum(scores[0], scores[1]), scores[2])
    es = [jnp.exp(s - m) for s in scores]
    tot = es[0] + es[1] + es[2]
    h_att = (ys1[0] * (es[0] / tot) + ys1[1] * (es[1] / tot)
             + ys1[2] * (es[2] / tot))

    cat = jnp.concatenate([h_att, xt0[...], xt1[...], xt2[...]], axis=1)
    o1 = jax.nn.relu(mm(cat, f1W) + f1b[...])
    o2 = jax.nn.relu(jnp.sum(o1 * f2W[...], axis=1, keepdims=True)
                     + f2b[0, 0])
    out[...] = o2


def _fused_dense(h1, h2, x, Wih0, Whh0, b0, Wih1, Whh1, b1v,
                 awW, awb, avW, avb, f1W, f1b, f2W, f2b):
    def row_spec(t):
        return pl.BlockSpec((_BLK, NFEAT), lambda i, t=t: (i + t * _NBLK, 0))

    full = lambda s: pl.BlockSpec(s, lambda i: tuple(0 for _ in s))
    in_specs = (
        [row_spec(t) for t in range(3)]
        + [row_spec(t) for t in range(3)]
        + [row_spec(t) for t in range(3)]
        + [full((4 * NHID, 2 * NHID)), full((4 * NHID, NHID)), full((1, 4 * NHID)),
           full((4 * NHID, NHID)), full((4 * NHID, NHID)), full((1, 4 * NHID)),
           full((NHID // 2, NHID)), full((1, NHID // 2)),
           full((1, NHID // 2)), full((1, 1)),
           full((NHID, NHID + WINDOW * NFEAT)), full((1, NHID)),
           full((1, NHID)), full((1, 1))]
    )
    return pl.pallas_call(
        _fused_dense_body,
        grid=(_NBLK,),
        in_specs=in_specs,
        out_specs=pl.BlockSpec((_BLK, 1), lambda i: (i, 0)),
        out_shape=jax.ShapeDtypeStruct((N_NODES, 1), jnp.float32),
    )(h1, h1, h1, h2, h2, h2, x, x, x,
      Wih0, Whh0, b0, Wih1, Whh1, b1v, awW, awb, avW, avb, f1W, f1b, f2W, f2b)


def _gcn_conv(x, edge_index, edge_weight, W, b, dinv):
    row, col = edge_index[0], edge_index[1]
    h = x @ W.T
    hp = dinv[:, None] * h
    s = jax.ops.segment_sum(hp[row] * edge_weight[:, None], col,
                            num_segments=N_TOTAL)
    return dinv[:, None] * s + dinv[:, None] ** 2 * h + b


def _batch_norm(z, g, b):
    m = jnp.mean(z, axis=0)
    v = jnp.var(z, axis=0)
    return (z - m) * lax.rsqrt(v + 1e-5) * g + b


def kernel(x, edge_index, edge_weight, W1, b1, W2, b2, bn1_g, bn1_b, bn2_g,
           bn2_b, Wih0, Whh0, bih0, bhh0, Wih1, Whh1, bih1, bhh1,
           attn_w_W, attn_w_b, attn_v_W, attn_v_b, fc1_W, fc1_b, fc2_W, fc2_b):
    col = edge_index[1]
    deg = jax.ops.segment_sum(edge_weight, col, num_segments=N_TOTAL) + 1.0
    dinv = lax.rsqrt(deg)

    h1 = jax.nn.relu(_gcn_conv(x, edge_index, edge_weight, W1, b1, dinv))
    h1 = _batch_norm(h1, bn1_g, bn1_b)
    h2 = jax.nn.relu(_gcn_conv(h1, edge_index, edge_weight, W2, b2, dinv))
    h2 = _batch_norm(h2, bn2_g, bn2_b)

    out = _fused_dense(
        h1, h2, x,
        Wih0, Whh0, (bih0 + bhh0)[None, :], Wih1, Whh1, (bih1 + bhh1)[None, :],
        attn_w_W, attn_w_b[None, :], attn_v_W, attn_v_b[None, :],
        fc1_W, fc1_b[None, :], fc2_W, fc2_b[None, :])
    return out.reshape(-1)
